# baseline (speedup 1.0000x reference)
import jax
import jax.numpy as jnp
from jax.experimental import pallas as pl
from jax.experimental.pallas import tpu as pltpu

B, G, E, H = 1024, 100, 64, 64
BB = 256
NT = B // BB
GC = 20
BC = 256


def _cell(x, h, c, W_ih, W_hh, b_ih, b_hh):
    mm1 = jax.lax.dot_general(x, W_ih, (((1,), (1,)), ((), ())))
    mm2 = jax.lax.dot_general(h, W_hh, (((1,), (1,)), ((), ())))
    gates = mm1 + mm2 + b_ih + b_hh
    i = jax.nn.sigmoid(gates[:, 0 * H:1 * H])
    f = jax.nn.sigmoid(gates[:, 1 * H:2 * H])
    g = jnp.tanh(gates[:, 2 * H:3 * H])
    o = jax.nn.sigmoid(gates[:, 3 * H:4 * H])
    c_new = f * c + i * g
    h_new = o * jnp.tanh(c_new)
    return h_new, c_new


def _ptr_kernel(state_ref, node_ref, aemb_ref,
                eWih_ref, eWhh_ref, ebih_ref, ebhh_ref,
                dWih_ref, dWhh_ref, dbih_ref, dbhh_ref,
                Wref_ref, Wq_ref, v_ref,
                acts_ref, w_ref,
                emb_s, ref_s, h_s, c_s, mask_s, din_s):
    t = pl.program_id(0)

    @pl.when(t < NT)
    def _encode():
        S = state_ref[...][:, 0, :]
        node = node_ref[...]
        for ci in range(G // GC):
            Sc = S[ci * GC:(ci + 1) * GC]
            iota_j = jax.lax.broadcasted_iota(
                jnp.int32, (GC, BB, G), 2).astype(jnp.float32)
            onehot = (Sc[:, :, None] == iota_j).astype(jnp.float32)
            emb = jax.lax.dot_general(onehot.reshape(GC * BB, G), node,
                                      (((1,), (0,)), ((), ())))
            emb_s[ci * GC:(ci + 1) * GC] = emb.reshape(GC, BB, E)

        eWih = eWih_ref[...]
        eWhh = eWhh_ref[...]
        ebih = ebih_ref[...]
        ebhh = ebhh_ref[...]
        Wref = Wref_ref[...]

        def enc_step(et, carry):
            h, c = carry
            h, c = _cell(emb_s[et], h, c, eWih, eWhh, ebih, ebhh)
            ref_s[t, et] = jax.lax.dot_general(Wref, h, (((1,), (1,)), ((), ())))
            return (h, c)

        h0 = jnp.zeros((BB, H), jnp.float32)
        c0 = jnp.zeros((BB, H), jnp.float32)
        h_enc, c_enc = jax.lax.fori_loop(0, G, enc_step, (h0, c0))
        h_s[t] = h_enc
        c_s[t] = c_enc
        mask_s[...] = jnp.ones((G, B), jnp.float32)
        din_s[...] = jnp.zeros((B, E), jnp.float32)

    @pl.when(t >= NT)
    def _decode():
        h_prev = h_s[...].reshape(B, H)
        c_prev = c_s[...].reshape(B, H)
        h, c = _cell(din_s[...], h_prev, c_prev,
                     dWih_ref[...], dWhh_ref[...], dbih_ref[...], dbhh_ref[...])
        h_s[...] = h.reshape(NT, BB, H)
        c_s[...] = c.reshape(NT, BB, H)
        qT = jax.lax.dot_general(Wq_ref[...], h, (((1,), (1,)), ((), ())))
        v_col = v_ref[...]

        es = []
        for ci in range(B // BC):
            ti, off = divmod(ci * BC, BB)
            rsl = ref_s[ti][:, :, off:off + BC]
            sc = jnp.tanh(rsl + qT[None, :, ci * BC:(ci + 1) * BC])
            es.append(jnp.sum(sc * v_col[None, :, :], axis=1))
        e = jnp.concatenate(es, axis=1)

        m = jnp.max(e, axis=0, keepdims=True)
        p = jnp.exp(e - m)
        probs = p / jnp.sum(p, axis=0, keepdims=True)
        mask = mask_s[...]
        masked = probs * mask
        wnorm = masked / jnp.sum(masked, axis=0, keepdims=True)
        mx = jnp.max(wnorm, axis=0, keepdims=True)
        iota0 = jax.lax.broadcasted_iota(jnp.int32, (G, B), 0)
        cand = jnp.where(wnorm == mx, iota0, G)
        topi = jnp.min(cand, axis=0, keepdims=True)

        w_ref[0] = wnorm
        acts_ref[...] = topi[None]
        mask_s[...] = jnp.where(iota0 == topi, 0.0, mask)
        onehot_a = (iota0 == topi).astype(jnp.float32)
        din_s[...] = jax.lax.dot_general(onehot_a, aemb_ref[...],
                                         (((0,), (0,)), ((), ())))


@jax.jit
def kernel(state, node_emb, action_emb, enc_W_ih, enc_W_hh, enc_b_ih,
           enc_b_hh, dec_W_ih, dec_W_hh, dec_b_ih, dec_b_hh, W_ref, W_q, v):
    state_t = state.T.reshape(G, 1, B)
    v_col = v.reshape(H, 1)
    ebih = enc_b_ih.reshape(1, 4 * H)
    ebhh = enc_b_hh.reshape(1, 4 * H)
    dbih = dec_b_ih.reshape(1, 4 * H)
    dbhh = dec_b_hh.reshape(1, 4 * H)

    full = lambda shape: pl.BlockSpec(shape, lambda t: tuple(0 for _ in shape))
    acts_tm, w_tm = pl.pallas_call(
        _ptr_kernel,
        grid=(NT + G,),
        in_specs=[
            pl.BlockSpec((G, 1, BB),
                         lambda t: (0, 0, jnp.minimum(t, NT - 1))),
            full((G, E)), full((G, E)),
            full((4 * H, E)), full((4 * H, H)), full((1, 4 * H)), full((1, 4 * H)),
            full((4 * H, E)), full((4 * H, H)), full((1, 4 * H)), full((1, 4 * H)),
            full((H, H)), full((H, H)), full((H, 1)),
        ],
        out_specs=[
            pl.BlockSpec((1, 1, B),
                         lambda t: (jnp.maximum(t - NT, 0), 0, 0)),
            pl.BlockSpec((1, G, B),
                         lambda t: (jnp.maximum(t - NT, 0), 0, 0)),
        ],
        out_shape=[
            jax.ShapeDtypeStruct((G, 1, B), jnp.int32),
            jax.ShapeDtypeStruct((G, G, B), jnp.float32),
        ],
        scratch_shapes=[
            pltpu.VMEM((G, BB, E), jnp.float32),
            pltpu.VMEM((NT, G, H, BB), jnp.float32),
            pltpu.VMEM((NT, BB, H), jnp.float32),
            pltpu.VMEM((NT, BB, H), jnp.float32),
            pltpu.VMEM((G, B), jnp.float32),
            pltpu.VMEM((B, E), jnp.float32),
        ],
    )(state_t, node_emb, action_emb, enc_W_ih, enc_W_hh, ebih, ebhh,
      dec_W_ih, dec_W_hh, dbih, dbhh, W_ref, W_q, v_col)

    actions = acts_tm[:, 0, :].T[:, :, None]
    attention_weights = jnp.transpose(w_tm, (2, 0, 1))
    return (actions, attention_weights)


# speedup vs baseline: 1.0793x; 1.0793x over previous
import jax
import jax.numpy as jnp
from jax.experimental import pallas as pl
from jax.experimental.pallas import tpu as pltpu

B, G, E, H = 1024, 100, 64, 64
BB = 512
NT = B // BB
GC = 20
BC = 256


def _cell(x, h, c, W_ih, W_hh, b_ih, b_hh):
    mm1 = jax.lax.dot_general(x, W_ih, (((1,), (1,)), ((), ())))
    mm2 = jax.lax.dot_general(h, W_hh, (((1,), (1,)), ((), ())))
    gates = mm1 + mm2 + b_ih + b_hh
    i = jax.nn.sigmoid(gates[:, 0 * H:1 * H])
    f = jax.nn.sigmoid(gates[:, 1 * H:2 * H])
    g = jnp.tanh(gates[:, 2 * H:3 * H])
    o = jax.nn.sigmoid(gates[:, 3 * H:4 * H])
    c_new = f * c + i * g
    h_new = o * jnp.tanh(c_new)
    return h_new, c_new


def _ptr_kernel(state_ref, node_ref, aemb_ref,
                eWih_ref, eWhh_ref, ebih_ref, ebhh_ref,
                dWih_ref, dWhh_ref, dbih_ref, dbhh_ref,
                Wref_ref, Wq_ref, v_ref,
                acts_ref, w_ref,
                emb_s, ref_s, h_s, c_s, mask_s, din_s):
    t = pl.program_id(0)

    @pl.when(t < NT)
    def _encode():
        S = state_ref[...][:, 0, :]
        node = node_ref[...]
        eWih = eWih_ref[...]
        eWhh = eWhh_ref[...]
        ebih = ebih_ref[...]
        ebhh = ebhh_ref[...]
        Wref = Wref_ref[...]

        h = jnp.zeros((BB, H), jnp.float32)
        c = jnp.zeros((BB, H), jnp.float32)
        for ci in range(G // GC):
            Sc = S[ci * GC:(ci + 1) * GC]
            iota_j = jax.lax.broadcasted_iota(
                jnp.int32, (GC, BB, G), 2).astype(jnp.float32)
            onehot = (Sc[:, :, None] == iota_j).astype(jnp.float32)
            emb = jax.lax.dot_general(onehot.reshape(GC * BB, G), node,
                                      (((1,), (0,)), ((), ())))
            emb_s[...] = emb.reshape(GC, BB, E)

            def enc_step(k, carry):
                h, c = carry
                h, c = _cell(emb_s[k], h, c, eWih, eWhh, ebih, ebhh)
                ref_s[t, ci * GC + k] = jax.lax.dot_general(
                    Wref, h, (((1,), (1,)), ((), ())))
                return (h, c)

            h, c = jax.lax.fori_loop(0, GC, enc_step, (h, c))
        h_s[t] = h
        c_s[t] = c
        mask_s[...] = jnp.ones((G, B), jnp.float32)
        din_s[...] = jnp.zeros((B, E), jnp.float32)

    @pl.when(t >= NT)
    def _decode():
        h_prev = h_s[...].reshape(B, H)
        c_prev = c_s[...].reshape(B, H)
        h, c = _cell(din_s[...], h_prev, c_prev,
                     dWih_ref[...], dWhh_ref[...], dbih_ref[...], dbhh_ref[...])
        h_s[...] = h.reshape(NT, BB, H)
        c_s[...] = c.reshape(NT, BB, H)
        qT = jax.lax.dot_general(Wq_ref[...], h, (((1,), (1,)), ((), ())))
        v_col = v_ref[...]

        es = []
        for ci in range(B // BC):
            ti, off = divmod(ci * BC, BB)
            rsl = ref_s[ti][:, :, off:off + BC]
            sc = jnp.tanh(rsl + qT[None, :, ci * BC:(ci + 1) * BC])
            es.append(jnp.sum(sc * v_col[None, :, :], axis=1))
        e = jnp.concatenate(es, axis=1)

        m = jnp.max(e, axis=0, keepdims=True)
        p = jnp.exp(e - m)
        probs = p / jnp.sum(p, axis=0, keepdims=True)
        mask = mask_s[...]
        masked = probs * mask
        wnorm = masked / jnp.sum(masked, axis=0, keepdims=True)
        mx = jnp.max(wnorm, axis=0, keepdims=True)
        iota0 = jax.lax.broadcasted_iota(jnp.int32, (G, B), 0)
        cand = jnp.where(wnorm == mx, iota0, G)
        topi = jnp.min(cand, axis=0, keepdims=True)

        w_ref[0] = wnorm
        acts_ref[...] = topi[None]
        mask_s[...] = jnp.where(iota0 == topi, 0.0, mask)
        onehot_a = (iota0 == topi).astype(jnp.float32)
        din_s[...] = jax.lax.dot_general(onehot_a, aemb_ref[...],
                                         (((0,), (0,)), ((), ())))


@jax.jit
def kernel(state, node_emb, action_emb, enc_W_ih, enc_W_hh, enc_b_ih,
           enc_b_hh, dec_W_ih, dec_W_hh, dec_b_ih, dec_b_hh, W_ref, W_q, v):
    state_t = state.T.reshape(G, 1, B)
    v_col = v.reshape(H, 1)
    ebih = enc_b_ih.reshape(1, 4 * H)
    ebhh = enc_b_hh.reshape(1, 4 * H)
    dbih = dec_b_ih.reshape(1, 4 * H)
    dbhh = dec_b_hh.reshape(1, 4 * H)

    full = lambda shape: pl.BlockSpec(shape, lambda t: tuple(0 for _ in shape))
    acts_tm, w_tm = pl.pallas_call(
        _ptr_kernel,
        grid=(NT + G,),
        in_specs=[
            pl.BlockSpec((G, 1, BB),
                         lambda t: (0, 0, jnp.minimum(t, NT - 1))),
            full((G, E)), full((G, E)),
            full((4 * H, E)), full((4 * H, H)), full((1, 4 * H)), full((1, 4 * H)),
            full((4 * H, E)), full((4 * H, H)), full((1, 4 * H)), full((1, 4 * H)),
            full((H, H)), full((H, H)), full((H, 1)),
        ],
        out_specs=[
            pl.BlockSpec((1, 1, B),
                         lambda t: (jnp.maximum(t - NT, 0), 0, 0)),
            pl.BlockSpec((1, G, B),
                         lambda t: (jnp.maximum(t - NT, 0), 0, 0)),
        ],
        out_shape=[
            jax.ShapeDtypeStruct((G, 1, B), jnp.int32),
            jax.ShapeDtypeStruct((G, G, B), jnp.float32),
        ],
        scratch_shapes=[
            pltpu.VMEM((GC, BB, E), jnp.float32),
            pltpu.VMEM((NT, G, H, BB), jnp.float32),
            pltpu.VMEM((NT, BB, H), jnp.float32),
            pltpu.VMEM((NT, BB, H), jnp.float32),
            pltpu.VMEM((G, B), jnp.float32),
            pltpu.VMEM((B, E), jnp.float32),
        ],
    )(state_t, node_emb, action_emb, enc_W_ih, enc_W_hh, ebih, ebhh,
      dec_W_ih, dec_W_hh, dbih, dbhh, W_ref, W_q, v_col)

    actions = acts_tm[:, 0, :].T[:, :, None]
    attention_weights = jnp.transpose(w_tm, (2, 0, 1))
    return (actions, attention_weights)


# speedup vs baseline: 1.0873x; 1.0074x over previous
import jax
import jax.numpy as jnp
from jax.experimental import pallas as pl
from jax.experimental.pallas import tpu as pltpu

B, G, E, H = 1024, 100, 64, 64
BB = 512
NT = B // BB
GC = 20
BC = 512


def _cell(x, h, c, W_ih, W_hh, b_ih, b_hh):
    mm1 = jax.lax.dot_general(x, W_ih, (((1,), (1,)), ((), ())))
    mm2 = jax.lax.dot_general(h, W_hh, (((1,), (1,)), ((), ())))
    gates = mm1 + mm2 + b_ih + b_hh
    i = jax.nn.sigmoid(gates[:, 0 * H:1 * H])
    f = jax.nn.sigmoid(gates[:, 1 * H:2 * H])
    g = jnp.tanh(gates[:, 2 * H:3 * H])
    o = jax.nn.sigmoid(gates[:, 3 * H:4 * H])
    c_new = f * c + i * g
    h_new = o * jnp.tanh(c_new)
    return h_new, c_new


def _ptr_kernel(state_ref, node_ref, aemb_ref,
                eWih_ref, eWhh_ref, ebih_ref, ebhh_ref,
                dWih_ref, dWhh_ref, dbih_ref, dbhh_ref,
                Wref_ref, Wq_ref, v_ref,
                acts_ref, w_ref,
                emb_s, ref_s, h_s, c_s, mask_s, din_s):
    t = pl.program_id(0)

    @pl.when(t < NT)
    def _encode():
        S = state_ref[...][:, 0, :]
        node = node_ref[...]
        eWih = eWih_ref[...]
        eWhh = eWhh_ref[...]
        ebih = ebih_ref[...]
        ebhh = ebhh_ref[...]
        Wref = Wref_ref[...]

        h = jnp.zeros((BB, H), jnp.float32)
        c = jnp.zeros((BB, H), jnp.float32)
        for ci in range(G // GC):
            Sc = S[ci * GC:(ci + 1) * GC]
            iota_j = jax.lax.broadcasted_iota(
                jnp.int32, (GC, BB, G), 2).astype(jnp.float32)
            onehot = (Sc[:, :, None] == iota_j).astype(jnp.float32)
            emb = jax.lax.dot_general(onehot.reshape(GC * BB, G), node,
                                      (((1,), (0,)), ((), ())))
            emb_s[...] = emb.reshape(GC, BB, E)

            def enc_step(k, carry):
                h, c = carry
                h, c = _cell(emb_s[k], h, c, eWih, eWhh, ebih, ebhh)
                ref_s[t, ci * GC + k] = jax.lax.dot_general(
                    Wref, h, (((1,), (1,)), ((), ())))
                return (h, c)

            h, c = jax.lax.fori_loop(0, GC, enc_step, (h, c))
        h_s[t] = h
        c_s[t] = c
        mask_s[...] = jnp.ones((G, B), jnp.float32)
        din_s[...] = jnp.zeros((B, E), jnp.float32)

    @pl.when(t >= NT)
    def _decode():
        h_prev = h_s[...].reshape(B, H)
        c_prev = c_s[...].reshape(B, H)
        h, c = _cell(din_s[...], h_prev, c_prev,
                     dWih_ref[...], dWhh_ref[...], dbih_ref[...], dbhh_ref[...])
        h_s[...] = h.reshape(NT, BB, H)
        c_s[...] = c.reshape(NT, BB, H)
        qT = jax.lax.dot_general(Wq_ref[...], h, (((1,), (1,)), ((), ())))
        v_col = v_ref[...]

        es = []
        for ci in range(B // BC):
            ti, off = divmod(ci * BC, BB)
            rsl = ref_s[ti][:, :, off:off + BC]
            sc = jnp.tanh(rsl + qT[None, :, ci * BC:(ci + 1) * BC])
            es.append(jnp.sum(sc * v_col[None, :, :], axis=1))
        e = jnp.concatenate(es, axis=1)

        m = jnp.max(e, axis=0, keepdims=True)
        p = jnp.exp(e - m)
        probs = p / jnp.sum(p, axis=0, keepdims=True)
        mask = mask_s[...]
        masked = probs * mask
        wnorm = masked / jnp.sum(masked, axis=0, keepdims=True)
        mx = jnp.max(wnorm, axis=0, keepdims=True)
        iota0 = jax.lax.broadcasted_iota(jnp.int32, (G, B), 0)
        cand = jnp.where(wnorm == mx, iota0, G)
        topi = jnp.min(cand, axis=0, keepdims=True)

        w_ref[0] = wnorm
        acts_ref[...] = topi[None]
        mask_s[...] = jnp.where(iota0 == topi, 0.0, mask)
        onehot_a = (iota0 == topi).astype(jnp.float32)
        din_s[...] = jax.lax.dot_general(onehot_a, aemb_ref[...],
                                         (((0,), (0,)), ((), ())))


@jax.jit
def kernel(state, node_emb, action_emb, enc_W_ih, enc_W_hh, enc_b_ih,
           enc_b_hh, dec_W_ih, dec_W_hh, dec_b_ih, dec_b_hh, W_ref, W_q, v):
    state_t = state.T.reshape(G, 1, B)
    v_col = v.reshape(H, 1)
    ebih = enc_b_ih.reshape(1, 4 * H)
    ebhh = enc_b_hh.reshape(1, 4 * H)
    dbih = dec_b_ih.reshape(1, 4 * H)
    dbhh = dec_b_hh.reshape(1, 4 * H)

    full = lambda shape: pl.BlockSpec(shape, lambda t: tuple(0 for _ in shape))
    acts_tm, w_tm = pl.pallas_call(
        _ptr_kernel,
        grid=(NT + G,),
        in_specs=[
            pl.BlockSpec((G, 1, BB),
                         lambda t: (0, 0, jnp.minimum(t, NT - 1))),
            full((G, E)), full((G, E)),
            full((4 * H, E)), full((4 * H, H)), full((1, 4 * H)), full((1, 4 * H)),
            full((4 * H, E)), full((4 * H, H)), full((1, 4 * H)), full((1, 4 * H)),
            full((H, H)), full((H, H)), full((H, 1)),
        ],
        out_specs=[
            pl.BlockSpec((1, 1, B),
                         lambda t: (jnp.maximum(t - NT, 0), 0, 0)),
            pl.BlockSpec((1, G, B),
                         lambda t: (jnp.maximum(t - NT, 0), 0, 0)),
        ],
        out_shape=[
            jax.ShapeDtypeStruct((G, 1, B), jnp.int32),
            jax.ShapeDtypeStruct((G, G, B), jnp.float32),
        ],
        scratch_shapes=[
            pltpu.VMEM((GC, BB, E), jnp.float32),
            pltpu.VMEM((NT, G, H, BB), jnp.float32),
            pltpu.VMEM((NT, BB, H), jnp.float32),
            pltpu.VMEM((NT, BB, H), jnp.float32),
            pltpu.VMEM((G, B), jnp.float32),
            pltpu.VMEM((B, E), jnp.float32),
        ],
    )(state_t, node_emb, action_emb, enc_W_ih, enc_W_hh, ebih, ebhh,
      dec_W_ih, dec_W_hh, dbih, dbhh, W_ref, W_q, v_col)

    actions = acts_tm[:, 0, :].T[:, :, None]
    attention_weights = jnp.transpose(w_tm, (2, 0, 1))
    return (actions, attention_weights)
